# baseline (speedup 1.0000x reference)
import functools

import jax
import jax.numpy as jnp
from jax import lax
from jax.experimental import pallas as pl
from jax.experimental.pallas import tpu as pltpu
from jax.experimental.pallas import tpu_sc as plsc

N = 10000
E = 320000
D = 128
ALPHA = 0.2

NC = 2
NS = 16
NW = NC * NS
EPT = E // NW
CHUNK = 40
NCHUNK = EPT // CHUNK
ROWS_PER_SUB = N // NS
SLAB = 25
ACC_W = 144

_f32 = jnp.float32


def _splat_lane(v, lane):
    dn = lax.GatherDimensionNumbers(
        offset_dims=(), collapsed_slice_dims=(0,), start_index_map=(0,))
    idx = jnp.full((16, 1), lane, jnp.int32)
    return lax.gather(v, idx, dn, (1,),
                      mode=lax.GatherScatterMode.PROMISE_IN_BOUNDS)


def _prep_body(x_ref, wfc_ref, wattn_ref, h_ref, asrc_ref, adst_ref):
    xb = x_ref[...]
    h = lax.dot_general(xb, wfc_ref[...], (((1,), (1,)), ((), ())),
                        preferred_element_type=_f32)
    wa = wattn_ref[...]
    a1 = wa[:, :D]
    a2 = wa[:, D:]
    h_ref[...] = h
    asrc_ref[...] = jnp.sum(h * a1, axis=1, keepdims=True)
    adst_ref[...] = jnp.sum(h * a2, axis=1, keepdims=True)


_PREP_B = 400


def _prep(x, W_fc, W_attn):
    grid = N // _PREP_B
    return pl.pallas_call(
        _prep_body,
        grid=(grid,),
        in_specs=[
            pl.BlockSpec((_PREP_B, D), lambda i: (i, 0)),
            pl.BlockSpec((D, D), lambda i: (0, 0)),
            pl.BlockSpec((1, 2 * D), lambda i: (0, 0)),
        ],
        out_specs=[
            pl.BlockSpec((_PREP_B, D), lambda i: (i, 0)),
            pl.BlockSpec((_PREP_B, 1), lambda i: (i, 0)),
            pl.BlockSpec((_PREP_B, 1), lambda i: (i, 0)),
        ],
        out_shape=[
            jax.ShapeDtypeStruct((N, D), _f32),
            jax.ShapeDtypeStruct((N, 1), _f32),
            jax.ShapeDtypeStruct((N, 1), _f32),
        ],
    )(x, W_fc, W_attn)


WIN = 50
NWIN = NCHUNK // WIN
GROUPS = ((0, 0), (16, 0), (24, 8))


def _edge_body(h_hbm, asrc_hbm, adst_hbm, src_hbm, dst_hbm, out_hbm,
               srcw, dstw, asbuf0, adbuf0, gbuf0, sbuf0,
               asbuf1, adbuf1, gbuf1, sbuf1, bounce, gsem, ssem, acc):
    asbufs, adbufs = (asbuf0, asbuf1), (adbuf0, adbuf1)
    gbufs, sbufs = (gbuf0, gbuf1), (sbuf0, sbuf1)
    cid = lax.axis_index("c")
    sid = lax.axis_index("s")
    wid = cid * NS + sid

    zero16 = jnp.zeros((16,), _f32)

    @pl.loop(0, SLAB)
    def _zb(i):
        for j in range(0, ACC_W, 16):
            bounce[i, pl.ds(j, 16)] = zero16

    @pl.loop(0, CHUNK)
    def _zs(i):
        for j in range(0, ACC_W, 16):
            sbuf0[i, pl.ds(j, 16)] = zero16
            sbuf1[i, pl.ds(j, 16)] = zero16

    @pl.loop(0, ROWS_PER_SUB // SLAB)
    def _zacc(j):
        pltpu.sync_copy(bounce, acc.at[pl.ds(sid * ROWS_PER_SUB + j * SLAB, SLAB)])

    plsc.subcore_barrier()

    iota16 = lax.iota(jnp.int32, 16)
    col_w = jnp.full((16,), D, jnp.int32)

    def _fire_gathers(c, b):
        pltpu.async_copy(h_hbm.at[dstw.at[c]], gbufs[b], gsem)
        pltpu.async_copy(asrc_hbm.at[srcw.at[c]], asbufs[b], gsem)
        pltpu.async_copy(adst_hbm.at[dstw.at[c]], adbufs[b], gsem)

    def _wait_gathers(c, b):
        pltpu.make_async_copy(h_hbm.at[dstw.at[c]], gbufs[b], gsem).wait()
        pltpu.make_async_copy(asrc_hbm.at[srcw.at[c]], asbufs[b], gsem).wait()
        pltpu.make_async_copy(adst_hbm.at[dstw.at[c]], adbufs[b], gsem).wait()

    @pl.loop(0, NWIN)
    def _win(w):
        @pl.when(w > 0)
        def _():
            pltpu.make_async_copy(sbuf0, acc.at[srcw.at[0]], ssem).wait()
            pltpu.make_async_copy(sbuf1, acc.at[srcw.at[0]], ssem).wait()

        pltpu.sync_copy(src_hbm.at[wid, pl.ds(w * WIN, WIN)], srcw)
        pltpu.sync_copy(dst_hbm.at[wid, pl.ds(w * WIN, WIN)], dstw)

        _fire_gathers(0, 0)

        @pl.loop(0, WIN)
        def _chunk(c):
            par = lax.rem(c, 2)
            for b in range(2):
                @pl.when(par == b)
                def _():
                    _wait_gathers(c, b)

                    @pl.when(c + 1 < WIN)
                    def _():
                        _fire_gathers(c + 1, 1 - b)

                    @pl.when(c > 1)
                    def _():
                        pltpu.make_async_copy(sbufs[b], acc.at[srcw.at[c]],
                                              ssem).wait()

                    gbuf, asbuf, adbuf = gbufs[b], asbufs[b], adbufs[b]
                    sbuf = sbufs[b]
                    for start, r0 in GROUPS:
                        s = (asbuf[pl.ds(start, 16)]
                             + adbuf[pl.ds(start, 16)])
                        leaky = jnp.where(s > 0, s, ALPHA * s)
                        w16 = jnp.exp(-leaky)
                        rowids = iota16 + start
                        plsc.store_scatter(sbuf, [rowids, col_w], w16)
                        for r in range(r0, 16):
                            row = start + r
                            wr = _splat_lane(w16, r)
                            for j in range(0, D, 16):
                                sbuf[row, pl.ds(j, 16)] = (
                                    gbuf[row, pl.ds(j, 16)] * wr)

                    pltpu.async_copy(sbuf, acc.at[srcw.at[c]], ssem, add=True)

    pltpu.make_async_copy(sbuf0, acc.at[srcw.at[0]], ssem).wait()
    pltpu.make_async_copy(sbuf1, acc.at[srcw.at[0]], ssem).wait()

    plsc.subcore_barrier()

    @pl.loop(0, ROWS_PER_SUB // SLAB)
    def _rb(j):
        base = sid * ROWS_PER_SUB + j * SLAB
        pltpu.sync_copy(acc.at[pl.ds(base, SLAB)], bounce)
        pltpu.sync_copy(bounce, out_hbm.at[cid, pl.ds(base, SLAB)])


_edge_kernel = pl.kernel(
    _edge_body,
    out_type=jax.ShapeDtypeStruct((NC, N, ACC_W), _f32),
    mesh=plsc.VectorSubcoreMesh(core_axis_name="c", subcore_axis_name="s"),
    compiler_params=pltpu.CompilerParams(use_tc_tiling_on_sc=False,
                                         needs_layout_passes=False),
    scratch_types=[
        pltpu.VMEM((WIN, CHUNK), jnp.int32),
        pltpu.VMEM((WIN, CHUNK), jnp.int32),
        pltpu.VMEM((CHUNK,), _f32),
        pltpu.VMEM((CHUNK,), _f32),
        pltpu.VMEM((CHUNK, D), _f32),
        pltpu.VMEM((CHUNK, ACC_W), _f32),
        pltpu.VMEM((CHUNK,), _f32),
        pltpu.VMEM((CHUNK,), _f32),
        pltpu.VMEM((CHUNK, D), _f32),
        pltpu.VMEM((CHUNK, ACC_W), _f32),
        pltpu.VMEM((SLAB, ACC_W), _f32),
        pltpu.SemaphoreType.DMA,
        pltpu.SemaphoreType.DMA,
        pltpu.VMEM_SHARED((N, ACC_W), _f32),
    ],
)


def _final_body(acc_ref, nn_ref, o_ref):
    a = acc_ref[0] + acc_ref[1]
    num = a[:, :D]
    den = a[:, D:D + 1]
    y = num / den * nn_ref[...]
    o_ref[...] = jnp.where(y > 0, y, jnp.exp(jnp.minimum(y, 0.0)) - 1.0)


_FIN_B = 400


def _final(acc, n_norm):
    grid = N // _FIN_B
    return pl.pallas_call(
        _final_body,
        grid=(grid,),
        in_specs=[
            pl.BlockSpec((NC, _FIN_B, ACC_W), lambda i: (0, i, 0)),
            pl.BlockSpec((_FIN_B, 1), lambda i: (i, 0)),
        ],
        out_specs=pl.BlockSpec((_FIN_B, D), lambda i: (i, 0)),
        out_shape=jax.ShapeDtypeStruct((N, D), _f32),
    )(acc, n_norm)


def kernel(x, edge_index, n_norm, W_fc, W_attn):
    h, asrc, adst = _prep(x, W_fc, W_attn)
    src = edge_index[0].reshape(NW, NCHUNK, CHUNK)
    dst = edge_index[1].reshape(NW, NCHUNK, CHUNK)
    acc = _edge_kernel(h, asrc.reshape(N), adst.reshape(N), src, dst)
    return _final(acc, n_norm)


# speedup vs baseline: 1.2546x; 1.2546x over previous
import jax
import jax.numpy as jnp
from jax import lax
from jax.experimental import pallas as pl
from jax.experimental.pallas import tpu as pltpu
from jax.experimental.pallas import tpu_sc as plsc

N = 10000
E = 320000
D = 128
ALPHA = 0.2

NC = 2
NS = 16
NW = NC * NS
EPT = E // NW
CHUNK = 80
NCHUNK = EPT // CHUNK
ROWS_PER_SUB = N // NS
ACC_W = 144
NSLOT = 3

_f32 = jnp.float32


def _splat_lane(v, lane):
    dn = lax.GatherDimensionNumbers(
        offset_dims=(), collapsed_slice_dims=(0,), start_index_map=(0,))
    idx = jnp.full((16, 1), lane, jnp.int32)
    return lax.gather(v, idx, dn, (1,),
                      mode=lax.GatherScatterMode.PROMISE_IN_BOUNDS)


def _prep_body(x_ref, wfc_ref, wattn_ref, happ_ref, asrc_ref):
    xb = x_ref[...]
    h = lax.dot_general(xb, wfc_ref[...], (((1,), (1,)), ((), ())),
                        preferred_element_type=_f32)
    wa = wattn_ref[...]
    a1 = wa[:, :D]
    a2 = wa[:, D:]
    adst = jnp.sum(h * a2, axis=1, keepdims=True)
    happ_ref[...] = jnp.concatenate(
        [h, adst, jnp.zeros((h.shape[0], ACC_W - D - 1), _f32)], axis=1)
    asrc_ref[...] = jnp.sum(h * a1, axis=1, keepdims=True)


_PREP_B = 400


def _prep(x, W_fc, W_attn):
    grid = N // _PREP_B
    return pl.pallas_call(
        _prep_body,
        grid=(grid,),
        in_specs=[
            pl.BlockSpec((_PREP_B, D), lambda i: (i, 0)),
            pl.BlockSpec((D, D), lambda i: (0, 0)),
            pl.BlockSpec((1, 2 * D), lambda i: (0, 0)),
        ],
        out_specs=[
            pl.BlockSpec((_PREP_B, ACC_W), lambda i: (i, 0)),
            pl.BlockSpec((_PREP_B, 1), lambda i: (i, 0)),
        ],
        out_shape=[
            jax.ShapeDtypeStruct((N, ACC_W), _f32),
            jax.ShapeDtypeStruct((N, 1), _f32),
        ],
    )(x, W_fc, W_attn)


WIN = 25
NWIN = NCHUNK // WIN


def _edge_body(happ_hbm, asrc_hbm, src_hbm, dst_hbm, out_hbm,
               srcw, dstw, asb, buf, gsem, ssem, acc):
    cid = lax.axis_index("c")
    sid = lax.axis_index("s")
    wid = cid * NS + sid

    zero16 = jnp.zeros((16,), _f32)

    @pl.loop(0, CHUNK)
    def _zb(i):
        for j in range(0, ACC_W, 16):
            buf[0, i, pl.ds(j, 16)] = zero16

    base0 = sid * ROWS_PER_SUB

    @pl.loop(0, 7)
    def _zacc(j):
        pltpu.sync_copy(buf.at[0], acc.at[pl.ds(base0 + j * CHUNK, CHUNK)])

    pltpu.sync_copy(buf.at[0, pl.ds(0, 65)],
                    acc.at[pl.ds(base0 + 7 * CHUNK, 65)])

    plsc.subcore_barrier()

    iota16 = lax.iota(jnp.int32, 16)
    col_w = jnp.full((16,), D, jnp.int32)

    def _fire_gathers(c, slot):
        pltpu.async_copy(happ_hbm.at[dstw.at[c]], buf.at[slot], gsem)
        pltpu.async_copy(asrc_hbm.at[srcw.at[c]], asb.at[slot], gsem)

    def _wait_gathers(c, slot):
        pltpu.make_async_copy(happ_hbm.at[dstw.at[c]], buf.at[slot],
                              gsem).wait()
        pltpu.make_async_copy(asrc_hbm.at[srcw.at[c]], asb.at[slot],
                              gsem).wait()

    def _drain_scatter(c, slot):
        pltpu.make_async_copy(buf.at[slot], acc.at[srcw.at[c]], ssem).wait()

    @pl.loop(0, NWIN)
    def _win(w):
        @pl.when(w > 0)
        def _():
            for s in range(NSLOT):
                _drain_scatter(0, s)

        pltpu.sync_copy(src_hbm.at[wid, pl.ds(w * WIN, WIN)], srcw)
        pltpu.sync_copy(dst_hbm.at[wid, pl.ds(w * WIN, WIN)], dstw)

        _fire_gathers(0, 0)

        @pl.loop(0, WIN)
        def _chunk(c):
            slot = lax.rem(c, NSLOT)
            nslot = lax.rem(c + 1, NSLOT)

            _wait_gathers(c, slot)

            @pl.when(c + 1 < WIN)
            def _():
                @pl.when(c > 1)
                def _():
                    _drain_scatter(c, nslot)
                _fire_gathers(c + 1, nslot)

            for g in range(CHUNK // 16):
                start = g * 16
                rowids = iota16 + start
                adst16 = plsc.load_gather(buf.at[slot], [rowids, col_w])
                s = asb[slot, pl.ds(start, 16)] + adst16
                leaky = jnp.where(s > 0, s, ALPHA * s)
                w16 = jnp.exp(-leaky)
                plsc.store_scatter(buf.at[slot], [rowids, col_w], w16)
                for r in range(16):
                    row = start + r
                    wr = _splat_lane(w16, r)
                    for j in range(0, D, 16):
                        buf[slot, row, pl.ds(j, 16)] = (
                            buf[slot, row, pl.ds(j, 16)] * wr)

            pltpu.async_copy(buf.at[slot], acc.at[srcw.at[c]], ssem, add=True)

    for s in range(NSLOT):
        _drain_scatter(0, s)

    plsc.subcore_barrier()

    @pl.loop(0, 7)
    def _rb(j):
        base = base0 + j * CHUNK
        pltpu.sync_copy(acc.at[pl.ds(base, CHUNK)], buf.at[0])
        pltpu.sync_copy(buf.at[0], out_hbm.at[cid, pl.ds(base, CHUNK)])

    pltpu.sync_copy(acc.at[pl.ds(base0 + 7 * CHUNK, 65)],
                    buf.at[0, pl.ds(0, 65)])
    pltpu.sync_copy(buf.at[0, pl.ds(0, 65)],
                    out_hbm.at[cid, pl.ds(base0 + 7 * CHUNK, 65)])


_edge_kernel = pl.kernel(
    _edge_body,
    out_type=jax.ShapeDtypeStruct((NC, N, ACC_W), _f32),
    mesh=plsc.VectorSubcoreMesh(core_axis_name="c", subcore_axis_name="s"),
    compiler_params=pltpu.CompilerParams(use_tc_tiling_on_sc=False,
                                         needs_layout_passes=False),
    scratch_types=[
        pltpu.VMEM((WIN, CHUNK), jnp.int32),
        pltpu.VMEM((WIN, CHUNK), jnp.int32),
        pltpu.VMEM((NSLOT, CHUNK), _f32),
        pltpu.VMEM((NSLOT, CHUNK, ACC_W), _f32),
        pltpu.SemaphoreType.DMA,
        pltpu.SemaphoreType.DMA,
        pltpu.VMEM_SHARED((N, ACC_W), _f32),
    ],
)


def _final_body(acc_ref, nn_ref, o_ref):
    a = acc_ref[0] + acc_ref[1]
    num = a[:, :D]
    den = a[:, D:D + 1]
    y = num / den * nn_ref[...]
    o_ref[...] = jnp.where(y > 0, y, jnp.exp(jnp.minimum(y, 0.0)) - 1.0)


_FIN_B = 400


def _final(acc, n_norm):
    grid = N // _FIN_B
    return pl.pallas_call(
        _final_body,
        grid=(grid,),
        in_specs=[
            pl.BlockSpec((NC, _FIN_B, ACC_W), lambda i: (0, i, 0)),
            pl.BlockSpec((_FIN_B, 1), lambda i: (i, 0)),
        ],
        out_specs=pl.BlockSpec((_FIN_B, D), lambda i: (i, 0)),
        out_shape=jax.ShapeDtypeStruct((N, D), _f32),
    )(acc, n_norm)


def kernel(x, edge_index, n_norm, W_fc, W_attn):
    happ, asrc = _prep(x, W_fc, W_attn)
    src = edge_index[0].reshape(NW, NCHUNK, CHUNK)
    dst = edge_index[1].reshape(NW, NCHUNK, CHUNK)
    acc = _edge_kernel(happ, asrc.reshape(N), src, dst)
    return _final(acc, n_norm)


# speedup vs baseline: 1.3537x; 1.0789x over previous
import jax
import jax.numpy as jnp
from jax import lax
from jax.experimental import pallas as pl
from jax.experimental.pallas import tpu as pltpu
from jax.experimental.pallas import tpu_sc as plsc

N = 10000
E = 320000
D = 128
ALPHA = 0.2

NC = 2
NS = 16
NW = NC * NS
EPT = E // NW
CHUNK = 80
NCHUNK = EPT // CHUNK
ROWS_PER_SUB = N // NS
ACC_W = 144
DP = D // 2

_f32 = jnp.float32
_i32 = jnp.int32


def _splat_lane(v, lane):
    dn = lax.GatherDimensionNumbers(
        offset_dims=(), collapsed_slice_dims=(0,), start_index_map=(0,))
    idx = jnp.full((16, 1), lane, jnp.int32)
    return lax.gather(v, idx, dn, (1,),
                      mode=lax.GatherScatterMode.PROMISE_IN_BOUNDS)


def _prep_body(x_ref, wfc_ref, wattn_ref, hb_ref, asrc_ref, adst_ref):
    xb = x_ref[...]
    h = lax.dot_general(xb, wfc_ref[...], (((1,), (1,)), ((), ())),
                        preferred_element_type=_f32)
    wa = wattn_ref[...]
    a1 = wa[:, :D]
    a2 = wa[:, D:]
    asrc_ref[...] = jnp.sum(h * a1, axis=1, keepdims=True)
    adst_ref[...] = jnp.sum(h * a2, axis=1, keepdims=True)

    hr = h.astype(jnp.bfloat16).astype(_f32)
    hu = lax.bitcast_convert_type(hr, _i32)
    packs = []
    for g in range(D // 32):
        lo = hu[:, 32 * g:32 * g + 16]
        hi = hu[:, 32 * g + 16:32 * g + 32]
        packs.append(
            jnp.bitwise_or(
                lax.shift_right_logical(lo, 16),
                jnp.bitwise_and(hi, jnp.int32(-65536))))
    hb_ref[...] = jnp.concatenate(packs, axis=1)


_PREP_B = 400


def _prep(x, W_fc, W_attn):
    grid = N // _PREP_B
    return pl.pallas_call(
        _prep_body,
        grid=(grid,),
        in_specs=[
            pl.BlockSpec((_PREP_B, D), lambda i: (i, 0)),
            pl.BlockSpec((D, D), lambda i: (0, 0)),
            pl.BlockSpec((1, 2 * D), lambda i: (0, 0)),
        ],
        out_specs=[
            pl.BlockSpec((_PREP_B, DP), lambda i: (i, 0)),
            pl.BlockSpec((_PREP_B, 1), lambda i: (i, 0)),
            pl.BlockSpec((_PREP_B, 1), lambda i: (i, 0)),
        ],
        out_shape=[
            jax.ShapeDtypeStruct((N, DP), _i32),
            jax.ShapeDtypeStruct((N, 1), _f32),
            jax.ShapeDtypeStruct((N, 1), _f32),
        ],
    )(x, W_fc, W_attn)


WIN = 25
NWIN = NCHUNK // WIN


def _edge_body(hb_hbm, asrc_hbm, adst_hbm, src_hbm, dst_hbm, out_hbm,
               srcw, dstw, asb, adb, bb, sbuf, gsem, ssem, acc):
    cid = lax.axis_index("c")
    sid = lax.axis_index("s")
    wid = cid * NS + sid

    zero16 = jnp.zeros((16,), _f32)

    @pl.loop(0, CHUNK)
    def _zb(i):
        for j in range(0, ACC_W, 16):
            sbuf[0, i, pl.ds(j, 16)] = zero16
            sbuf[1, i, pl.ds(j, 16)] = zero16

    base0 = sid * ROWS_PER_SUB

    @pl.loop(0, 7)
    def _zacc(j):
        pltpu.sync_copy(sbuf.at[0], acc.at[pl.ds(base0 + j * CHUNK, CHUNK)])

    pltpu.sync_copy(sbuf.at[0, pl.ds(0, 65)],
                    acc.at[pl.ds(base0 + 7 * CHUNK, 65)])

    plsc.subcore_barrier()

    iota16 = lax.iota(jnp.int32, 16)
    col_w = jnp.full((16,), D, jnp.int32)
    himask = jnp.full((16,), -65536, _i32)

    def _fire_gathers(c, slot):
        pltpu.async_copy(hb_hbm.at[dstw.at[c]], bb.at[slot], gsem)
        pltpu.async_copy(asrc_hbm.at[srcw.at[c]], asb.at[slot], gsem)
        pltpu.async_copy(adst_hbm.at[dstw.at[c]], adb.at[slot], gsem)

    def _wait_gathers(c, slot):
        pltpu.make_async_copy(hb_hbm.at[dstw.at[c]], bb.at[slot], gsem).wait()
        pltpu.make_async_copy(asrc_hbm.at[srcw.at[c]], asb.at[slot],
                              gsem).wait()
        pltpu.make_async_copy(adst_hbm.at[dstw.at[c]], adb.at[slot],
                              gsem).wait()

    def _drain_scatter(c, slot):
        pltpu.make_async_copy(sbuf.at[slot], acc.at[srcw.at[c]], ssem).wait()

    @pl.loop(0, NWIN)
    def _win(w):
        @pl.when(w > 0)
        def _():
            _drain_scatter(0, 0)
            _drain_scatter(0, 1)

        pltpu.sync_copy(src_hbm.at[wid, pl.ds(w * WIN, WIN)], srcw)
        pltpu.sync_copy(dst_hbm.at[wid, pl.ds(w * WIN, WIN)], dstw)

        _fire_gathers(0, 0)

        @pl.loop(0, WIN)
        def _chunk(c):
            par = lax.rem(c, 2)
            for b in range(2):
                @pl.when(par == b)
                def _():
                    _wait_gathers(c, b)

                    @pl.when(c + 1 < WIN)
                    def _():
                        _fire_gathers(c + 1, 1 - b)

                    @pl.when(c > 1)
                    def _():
                        _drain_scatter(c, b)

                    for g in range(CHUNK // 16):
                        start = g * 16
                        s = (asb[b, pl.ds(start, 16)]
                             + adb[b, pl.ds(start, 16)])
                        leaky = jnp.where(s > 0, s, ALPHA * s)
                        w16 = jnp.exp(-leaky)
                        rowids = iota16 + start
                        plsc.store_scatter(sbuf.at[b], [rowids, col_w], w16)
                        for r in range(16):
                            row = start + r
                            wr = _splat_lane(w16, r)
                            for k in range(DP // 16):
                                v = bb[b, row, pl.ds(k * 16, 16)]
                                lo = plsc.bitcast(
                                    lax.shift_left(v, 16), _f32)
                                hi = plsc.bitcast(
                                    jnp.bitwise_and(v, himask), _f32)
                                sbuf[b, row, pl.ds(32 * k, 16)] = lo * wr
                                sbuf[b, row, pl.ds(32 * k + 16, 16)] = hi * wr

                    pltpu.async_copy(sbuf.at[b], acc.at[srcw.at[c]], ssem,
                                     add=True)

    _drain_scatter(0, 0)
    _drain_scatter(0, 1)

    plsc.subcore_barrier()

    @pl.loop(0, 7)
    def _rb(j):
        base = base0 + j * CHUNK
        pltpu.sync_copy(acc.at[pl.ds(base, CHUNK)], sbuf.at[0])
        pltpu.sync_copy(sbuf.at[0], out_hbm.at[cid, pl.ds(base, CHUNK)])

    pltpu.sync_copy(acc.at[pl.ds(base0 + 7 * CHUNK, 65)],
                    sbuf.at[0, pl.ds(0, 65)])
    pltpu.sync_copy(sbuf.at[0, pl.ds(0, 65)],
                    out_hbm.at[cid, pl.ds(base0 + 7 * CHUNK, 65)])


_edge_kernel = pl.kernel(
    _edge_body,
    out_type=jax.ShapeDtypeStruct((NC, N, ACC_W), _f32),
    mesh=plsc.VectorSubcoreMesh(core_axis_name="c", subcore_axis_name="s"),
    compiler_params=pltpu.CompilerParams(use_tc_tiling_on_sc=False,
                                         needs_layout_passes=False),
    scratch_types=[
        pltpu.VMEM((WIN, CHUNK), jnp.int32),
        pltpu.VMEM((WIN, CHUNK), jnp.int32),
        pltpu.VMEM((2, CHUNK), _f32),
        pltpu.VMEM((2, CHUNK), _f32),
        pltpu.VMEM((2, CHUNK, DP), _i32),
        pltpu.VMEM((2, CHUNK, ACC_W), _f32),
        pltpu.SemaphoreType.DMA,
        pltpu.SemaphoreType.DMA,
        pltpu.VMEM_SHARED((N, ACC_W), _f32),
    ],
)


def _final_body(acc_ref, nn_ref, o_ref):
    a = acc_ref[0] + acc_ref[1]
    num = a[:, :D]
    den = a[:, D:D + 1]
    y = num / den * nn_ref[...]
    o_ref[...] = jnp.where(y > 0, y, jnp.exp(jnp.minimum(y, 0.0)) - 1.0)


_FIN_B = 400


def _final(acc, n_norm):
    grid = N // _FIN_B
    return pl.pallas_call(
        _final_body,
        grid=(grid,),
        in_specs=[
            pl.BlockSpec((NC, _FIN_B, ACC_W), lambda i: (0, i, 0)),
            pl.BlockSpec((_FIN_B, 1), lambda i: (i, 0)),
        ],
        out_specs=pl.BlockSpec((_FIN_B, D), lambda i: (i, 0)),
        out_shape=jax.ShapeDtypeStruct((N, D), _f32),
    )(acc, n_norm)


def kernel(x, edge_index, n_norm, W_fc, W_attn):
    hb, asrc, adst = _prep(x, W_fc, W_attn)
    src = edge_index[0].reshape(NW, NCHUNK, CHUNK)
    dst = edge_index[1].reshape(NW, NCHUNK, CHUNK)
    acc = _edge_kernel(hb, asrc.reshape(N), adst.reshape(N), src, dst)
    return _final(acc, n_norm)


# speedup vs baseline: 1.4385x; 1.0627x over previous
import jax
import jax.numpy as jnp
from jax import lax
from jax.experimental import pallas as pl
from jax.experimental.pallas import tpu as pltpu
from jax.experimental.pallas import tpu_sc as plsc

N = 10000
E = 320000
D = 128
ALPHA = 0.2

NC = 2
NS = 16
NW = NC * NS
EPT = E // NW
CHUNK = 80
NCHUNK = EPT // CHUNK
ROWS_PER_SUB = N // NS
ACC_W = 128
DP = D // 2

_f32 = jnp.float32
_i32 = jnp.int32


def _splat_lane(v, lane):
    dn = lax.GatherDimensionNumbers(
        offset_dims=(), collapsed_slice_dims=(0,), start_index_map=(0,))
    idx = jnp.full((16, 1), lane, jnp.int32)
    return lax.gather(v, idx, dn, (1,),
                      mode=lax.GatherScatterMode.PROMISE_IN_BOUNDS)


def _prep_body(x_ref, wfc_ref, wattn_ref, hb_ref, asrc_ref, adst_ref):
    xb = x_ref[...]
    h = lax.dot_general(xb, wfc_ref[...], (((1,), (1,)), ((), ())),
                        preferred_element_type=_f32)
    wa = wattn_ref[...]
    a1 = wa[:, :D]
    a2 = wa[:, D:]
    asrc_ref[...] = jnp.sum(h * a1, axis=1, keepdims=True)
    adst_ref[...] = jnp.sum(h * a2, axis=1, keepdims=True)

    hr = h.astype(jnp.bfloat16).astype(_f32)
    hu = lax.bitcast_convert_type(hr, _i32)
    packs = []
    for g in range(D // 32):
        lo = hu[:, 32 * g:32 * g + 16]
        hi = hu[:, 32 * g + 16:32 * g + 32]
        packs.append(
            jnp.bitwise_or(
                lax.shift_right_logical(lo, 16),
                jnp.bitwise_and(hi, jnp.int32(-65536))))
    hb_ref[...] = jnp.concatenate(packs, axis=1)


_PREP_B = 400


def _prep(x, W_fc, W_attn):
    grid = N // _PREP_B
    return pl.pallas_call(
        _prep_body,
        grid=(grid,),
        in_specs=[
            pl.BlockSpec((_PREP_B, D), lambda i: (i, 0)),
            pl.BlockSpec((D, D), lambda i: (0, 0)),
            pl.BlockSpec((1, 2 * D), lambda i: (0, 0)),
        ],
        out_specs=[
            pl.BlockSpec((_PREP_B, DP), lambda i: (i, 0)),
            pl.BlockSpec((_PREP_B, 1), lambda i: (i, 0)),
            pl.BlockSpec((_PREP_B, 1), lambda i: (i, 0)),
        ],
        out_shape=[
            jax.ShapeDtypeStruct((N, DP), _i32),
            jax.ShapeDtypeStruct((N, 1), _f32),
            jax.ShapeDtypeStruct((N, 1), _f32),
        ],
    )(x, W_fc, W_attn)


WIN = 25
NWIN = NCHUNK // WIN


def _edge_body(hb_hbm, asrc_hbm, adst_hbm, src_hbm, dst_hbm, out_hbm, rs_hbm,
               srcw, dstw, asb, adb, bb, sbuf, rs, gsem, ssem, acc):
    cid = lax.axis_index("c")
    sid = lax.axis_index("s")
    wid = cid * NS + sid

    zero16 = jnp.zeros((16,), _f32)

    @pl.loop(0, CHUNK)
    def _zb(i):
        for j in range(0, ACC_W, 16):
            sbuf[0, i, pl.ds(j, 16)] = zero16

    @pl.loop(0, N // 16)
    def _zr(i):
        rs[pl.ds(i * 16, 16)] = zero16

    base0 = sid * ROWS_PER_SUB

    @pl.loop(0, 7)
    def _zacc(j):
        pltpu.sync_copy(sbuf.at[0], acc.at[pl.ds(base0 + j * CHUNK, CHUNK)])

    pltpu.sync_copy(sbuf.at[0, pl.ds(0, 65)],
                    acc.at[pl.ds(base0 + 7 * CHUNK, 65)])

    plsc.subcore_barrier()

    iota16 = lax.iota(jnp.int32, 16)
    himask = jnp.full((16,), -65536, _i32)

    def _fire_gathers(c, slot):
        pltpu.async_copy(hb_hbm.at[dstw.at[c]], bb.at[slot], gsem)
        pltpu.async_copy(asrc_hbm.at[srcw.at[c]], asb.at[slot], gsem)
        pltpu.async_copy(adst_hbm.at[dstw.at[c]], adb.at[slot], gsem)

    def _wait_gathers(c, slot):
        pltpu.make_async_copy(hb_hbm.at[dstw.at[c]], bb.at[slot], gsem).wait()
        pltpu.make_async_copy(asrc_hbm.at[srcw.at[c]], asb.at[slot],
                              gsem).wait()
        pltpu.make_async_copy(adst_hbm.at[dstw.at[c]], adb.at[slot],
                              gsem).wait()

    def _drain_scatter(c, slot):
        pltpu.make_async_copy(sbuf.at[slot], acc.at[srcw.at[c]], ssem).wait()

    @pl.loop(0, NWIN)
    def _win(w):
        @pl.when(w > 0)
        def _():
            _drain_scatter(0, 0)
            _drain_scatter(0, 1)

        pltpu.sync_copy(src_hbm.at[wid, pl.ds(w * WIN, WIN)], srcw)
        pltpu.sync_copy(dst_hbm.at[wid, pl.ds(w * WIN, WIN)], dstw)

        _fire_gathers(0, 0)

        @pl.loop(0, WIN)
        def _chunk(c):
            par = lax.rem(c, 2)
            for b in range(2):
                @pl.when(par == b)
                def _():
                    _wait_gathers(c, b)

                    @pl.when(c + 1 < WIN)
                    def _():
                        _fire_gathers(c + 1, 1 - b)

                    @pl.when(c > 1)
                    def _():
                        _drain_scatter(c, b)

                    for g in range(CHUNK // 16):
                        start = g * 16
                        s = (asb[b, pl.ds(start, 16)]
                             + adb[b, pl.ds(start, 16)])
                        leaky = jnp.where(s > 0, s, ALPHA * s)
                        w16 = jnp.exp(-leaky)
                        src16 = srcw[c, pl.ds(start, 16)]
                        plsc.addupdate_scatter(rs, [src16], w16)
                        for r in range(16):
                            row = start + r
                            wr = _splat_lane(w16, r)
                            for k in range(DP // 16):
                                v = bb[b, row, pl.ds(k * 16, 16)]
                                lo = plsc.bitcast(
                                    lax.shift_left(v, 16), _f32)
                                hi = plsc.bitcast(
                                    jnp.bitwise_and(v, himask), _f32)
                                sbuf[b, row, pl.ds(32 * k, 16)] = lo * wr
                                sbuf[b, row, pl.ds(32 * k + 16, 16)] = hi * wr

                    pltpu.async_copy(sbuf.at[b], acc.at[srcw.at[c]], ssem,
                                     add=True)

    _drain_scatter(0, 0)
    _drain_scatter(0, 1)

    plsc.subcore_barrier()

    @pl.loop(0, 7)
    def _rb(j):
        base = base0 + j * CHUNK
        pltpu.sync_copy(acc.at[pl.ds(base, CHUNK)], sbuf.at[0])
        pltpu.sync_copy(sbuf.at[0], out_hbm.at[cid, pl.ds(base, CHUNK)])

    pltpu.sync_copy(acc.at[pl.ds(base0 + 7 * CHUNK, 65)],
                    sbuf.at[0, pl.ds(0, 65)])
    pltpu.sync_copy(sbuf.at[0, pl.ds(0, 65)],
                    out_hbm.at[cid, pl.ds(base0 + 7 * CHUNK, 65)])
    pltpu.sync_copy(rs, rs_hbm.at[wid])


_edge_kernel = pl.kernel(
    _edge_body,
    out_type=[jax.ShapeDtypeStruct((NC, N, ACC_W), _f32),
              jax.ShapeDtypeStruct((NW, N), _f32)],
    mesh=plsc.VectorSubcoreMesh(core_axis_name="c", subcore_axis_name="s"),
    compiler_params=pltpu.CompilerParams(use_tc_tiling_on_sc=False,
                                         needs_layout_passes=False),
    scratch_types=[
        pltpu.VMEM((WIN, CHUNK), jnp.int32),
        pltpu.VMEM((WIN, CHUNK), jnp.int32),
        pltpu.VMEM((2, CHUNK), _f32),
        pltpu.VMEM((2, CHUNK), _f32),
        pltpu.VMEM((2, CHUNK, DP), _i32),
        pltpu.VMEM((2, CHUNK, ACC_W), _f32),
        pltpu.VMEM((N,), _f32),
        pltpu.SemaphoreType.DMA,
        pltpu.SemaphoreType.DMA,
        pltpu.VMEM_SHARED((N, ACC_W), _f32),
    ],
)


def _final_body(acc_ref, rs_ref, nn_ref, o_ref):
    num = acc_ref[0] + acc_ref[1]
    den = jnp.sum(rs_ref[0], axis=0)[:, None]
    y = num / den * nn_ref[...]
    o_ref[...] = jnp.where(y > 0, y, jnp.exp(jnp.minimum(y, 0.0)) - 1.0)


_FIN_B = 400


def _final(acc, rsall, n_norm):
    grid = N // _FIN_B
    return pl.pallas_call(
        _final_body,
        grid=(grid,),
        in_specs=[
            pl.BlockSpec((NC, _FIN_B, ACC_W), lambda i: (0, i, 0)),
            pl.BlockSpec((1, NW, _FIN_B), lambda i: (i, 0, 0)),
            pl.BlockSpec((_FIN_B, 1), lambda i: (i, 0)),
        ],
        out_specs=pl.BlockSpec((_FIN_B, D), lambda i: (i, 0)),
        out_shape=jax.ShapeDtypeStruct((N, D), _f32),
    )(acc, rsall, n_norm)


def kernel(x, edge_index, n_norm, W_fc, W_attn):
    hb, asrc, adst = _prep(x, W_fc, W_attn)
    src = edge_index[0].reshape(NW, NCHUNK, CHUNK)
    dst = edge_index[1].reshape(NW, NCHUNK, CHUNK)
    acc, rsall = _edge_kernel(hb, asrc.reshape(N), adst.reshape(N), src, dst)
    rsall = rsall.reshape(NW, N // _FIN_B, _FIN_B).transpose(1, 0, 2)
    return _final(acc, rsall, n_norm)
